# Initial kernel scaffold; baseline (speedup 1.0000x reference)
import jax
import jax.numpy as jnp
from jax.experimental import pallas as pl


def kernel(nodeTypes, edge_index, edge_attr, bs, emb, W1, root1, b1, W2, root2, b2, att_w, lin_w, lin_b):
    raise NotImplementedError("write your pallas kernel here")


# speedup vs baseline: 2.7032x; 2.7032x over previous
import functools

import jax
import jax.numpy as jnp
from jax import lax
from jax.experimental import pallas as pl
from jax.experimental.pallas import tpu as pltpu
from jax.experimental.pallas import tpu_sc as plsc

N = 10000
E = 320000
R = 16
G = 256
D = 128
NP = 12288
NR = 10112
EP = 323584
EPT = EP // 32
C = NR * R
CPT = C // 16
ART = NR // 16
DUMMY_DST = 10100

_mesh = plsc.VectorSubcoreMesh(core_axis_name="c", subcore_axis_name="s")
f32 = jnp.float32
i32 = jnp.int32


@functools.partial(
    pl.kernel,
    out_type=(
        jax.ShapeDtypeStruct((NP, D), f32),
        jax.ShapeDtypeStruct((EP,), i32),
        jax.ShapeDtypeStruct((EP,), i32),
        jax.ShapeDtypeStruct((2, C), f32),
    ),
    mesh=_mesh,
    scratch_types=[
        pltpu.VMEM((128,), i32),
        pltpu.VMEM((128, D), f32),
        pltpu.VMEM((128,), i32),
        pltpu.VMEM((128,), i32),
        pltpu.VMEM((128,), i32),
        pltpu.VMEM((128,), i32),
        pltpu.VMEM((128,), i32),
        pltpu.VMEM((128,), f32),
        pltpu.VMEM((CPT,), f32),
        pltpu.VMEM_SHARED((C,), f32),
        pltpu.SemaphoreType.DMA,
    ],
)
def _sc_prep(nt_hbm, emb_hbm, src_hbm, dst_hbm, et_hbm,
             x_hbm, rowid_hbm, comb_hbm, cntp_hbm,
             nidx, xrows, sbuf, dbuf, ebuf, rbuf, cbuf, ones, zb, cnt_sh, sem):
    c = lax.axis_index("c")
    s = lax.axis_index("s")
    wid = s * 2 + c

    for j in range(8):
        ones[pl.ds(j * 16, 16)] = jnp.full((16,), 1.0, f32)

    def zinit(j, _):
        zb[pl.ds(j * 16, 16)] = jnp.zeros((16,), f32)
        return 0
    lax.fori_loop(0, CPT // 16, zinit, 0)
    pltpu.sync_copy(zb, cnt_sh.at[pl.ds(s * CPT, CPT)])
    plsc.subcore_barrier()

    for k in range(3):
        nb = wid * 384 + k * 128
        pltpu.sync_copy(nt_hbm.at[pl.ds(nb, 128)], nidx)
        pltpu.async_copy(emb_hbm.at[nidx], xrows, sem).wait()
        pltpu.sync_copy(xrows, x_hbm.at[pl.ds(nb, 128)])

    def chunk(k, _):
        eb = wid * EPT + k * 128
        pltpu.sync_copy(src_hbm.at[pl.ds(eb, 128)], sbuf)
        pltpu.sync_copy(dst_hbm.at[pl.ds(eb, 128)], dbuf)
        pltpu.sync_copy(et_hbm.at[pl.ds(eb, 128)], ebuf)
        for j in range(8):
            sl = pl.ds(j * 16, 16)
            tv = ebuf[sl]
            rbuf[sl] = sbuf[sl] * R + tv
            cbuf[sl] = dbuf[sl] * R + tv
        pltpu.sync_copy(rbuf, rowid_hbm.at[pl.ds(eb, 128)])
        pltpu.sync_copy(cbuf, comb_hbm.at[pl.ds(eb, 128)])
        pltpu.sync_copy(ones, cnt_sh.at[cbuf], add=True)
        return 0
    lax.fori_loop(0, EPT // 128, chunk, 0)

    plsc.subcore_barrier()
    pltpu.sync_copy(cnt_sh.at[pl.ds(s * CPT, CPT)],
                    cntp_hbm.at[c, pl.ds(s * CPT, CPT)])


@functools.partial(
    pl.kernel,
    out_type=jax.ShapeDtypeStruct((2, NR, D), f32),
    mesh=_mesh,
    scratch_types=[
        pltpu.VMEM((128,), i32),
        pltpu.VMEM((128,), i32),
        pltpu.VMEM((128,), i32),
        pltpu.VMEM((128, D), f32),
        pltpu.VMEM((144,), f32),
        pltpu.VMEM((8, D), f32),
        pltpu.VMEM_SHARED((NR, D), f32),
        pltpu.SemaphoreType.DMA,
        pltpu.SemaphoreType.DMA,
    ],
)
def _sc_layer(hrows_hbm, rowid_hbm, comb_hbm, dst_hbm, norm_hbm,
              aggp_hbm,
              idxg, idxn, idxs, rows, nrm, zb, agg_sh, sem, sem2):
    c = lax.axis_index("c")
    s = lax.axis_index("s")
    wid = s * 2 + c

    for i in range(8):
        for j in range(8):
            zb[i, pl.ds(j * 16, 16)] = jnp.zeros((16,), f32)

    def zcopy(i, _):
        pltpu.sync_copy(zb, agg_sh.at[pl.ds(s * ART + i * 8, 8)])
        return 0
    lax.fori_loop(0, ART // 8, zcopy, 0)
    plsc.subcore_barrier()

    def chunk(k, _):
        eb = wid * EPT + k * 128
        pltpu.sync_copy(rowid_hbm.at[pl.ds(eb, 128)], idxg)
        pltpu.sync_copy(comb_hbm.at[pl.ds(eb, 128)], idxn)
        pltpu.sync_copy(dst_hbm.at[pl.ds(eb, 128)], idxs)
        gcp = pltpu.async_copy(hrows_hbm.at[idxg], rows, sem)
        pltpu.async_copy(norm_hbm.at[idxn], nrm.at[pl.ds(0, 128)], sem2).wait()
        gcp.wait()

        def edge(e, _):
            sc = jnp.full((16,), nrm[pl.ds(e, 16)][0], f32)
            for cc in range(8):
                sl = pl.ds(cc * 16, 16)
                rows[e, sl] = rows[e, sl] * sc
            return 0
        lax.fori_loop(0, 128, edge, 0)
        pltpu.sync_copy(rows, agg_sh.at[idxs], add=True)
        return 0
    lax.fori_loop(0, EPT // 128, chunk, 0)

    plsc.subcore_barrier()
    pltpu.sync_copy(agg_sh.at[pl.ds(s * ART, ART)],
                    aggp_hbm.at[c, pl.ds(s * ART, ART)])


def _norm_body(cnt_ref, norm_ref):
    cnt = cnt_ref[0] + cnt_ref[1]
    norm_ref[...] = 1.0 / jnp.maximum(cnt, 1.0)


_tc_norm = pl.pallas_call(
    _norm_body,
    out_shape=jax.ShapeDtypeStruct((C // 128, 128), f32),
)


def _mm_body(x_ref, w_ref, o_ref):
    o_ref[...] = jnp.dot(x_ref[...], w_ref[...], preferred_element_type=f32)


_tc_h1 = pl.pallas_call(
    _mm_body,
    grid=(NR // 128,),
    in_specs=[pl.BlockSpec((128, D), lambda i: (i, 0)),
              pl.BlockSpec((D, R * D), lambda i: (0, 0))],
    out_specs=pl.BlockSpec((128, R * D), lambda i: (i, 0)),
    out_shape=jax.ShapeDtypeStruct((NR, R * D), f32),
)


def _combine_body(x_ref, p0_ref, p1_ref, root_ref, b_ref, w2_ref,
                  h_ref, h2_ref):
    h = p0_ref[...] + p1_ref[...] + b_ref[...]
    h = h + jnp.dot(x_ref[...], root_ref[...], preferred_element_type=f32)
    h = jnp.maximum(h, 0.0)
    h_ref[...] = h
    h2_ref[...] = jnp.dot(h, w2_ref[...], preferred_element_type=f32)


_tc_combine = pl.pallas_call(
    _combine_body,
    grid=(NR // 128,),
    in_specs=[pl.BlockSpec((128, D), lambda i: (i, 0)),
              pl.BlockSpec((128, D), lambda i: (i, 0)),
              pl.BlockSpec((128, D), lambda i: (i, 0)),
              pl.BlockSpec((D, D), lambda i: (0, 0)),
              pl.BlockSpec((1, D), lambda i: (0, 0)),
              pl.BlockSpec((D, R * D), lambda i: (0, 0))],
    out_specs=[pl.BlockSpec((128, D), lambda i: (i, 0)),
               pl.BlockSpec((128, R * D), lambda i: (i, 0))],
    out_shape=[jax.ShapeDtypeStruct((NR, D), f32),
               jax.ShapeDtypeStruct((NR, R * D), f32)],
)


def _readout_body(p0_ref, p1_ref, h1_ref, root_ref, b_ref, awt_ref, bs_ref,
                  lw_ref, lb_ref, o_ref, h2_sc, s_sc):
    h2 = p0_ref[...] + p1_ref[...] + b_ref[...]
    h2 = h2 + jnp.dot(h1_ref[...], root_ref[...], preferred_element_type=f32)
    h2 = jnp.maximum(h2, 0.0)
    h2_sc[...] = h2

    neg_inf = jnp.float32(-jnp.inf)
    giota = lax.broadcasted_iota(i32, (G, 128), 0)
    col = lax.broadcasted_iota(i32, (1, 128), 1)

    def oh_mask(b):
        bsrow = bs_ref[pl.ds(b, 1), :]
        vmask = (b * 128 + col) < N
        return (giota == bsrow) & vmask

    def pass1(b, m):
        hblk = h2_sc[pl.ds(b * 128, 128), :]
        scb = lax.dot_general(awt_ref[...], hblk, (((1,), (1,)), ((), ())),
                              preferred_element_type=f32)
        s_sc[pl.ds(b, 1), :] = scb
        oh = oh_mask(b)
        vals = jnp.where(oh, jnp.broadcast_to(scb, (G, 128)), neg_inf)
        return jnp.maximum(m, jnp.max(vals, axis=1, keepdims=True))

    m = lax.fori_loop(0, NR // 128, pass1, jnp.full((G, 1), neg_inf, f32))
    m = jnp.where(jnp.isfinite(m), m, 0.0)

    def pass2(b, ssum):
        scb = s_sc[pl.ds(b, 1), :]
        oh = oh_mask(b)
        mrow = jnp.sum(jnp.where(oh, jnp.broadcast_to(m, (G, 128)), 0.0),
                       axis=0, keepdims=True)
        erow = jnp.where((b * 128 + col) < N, jnp.exp(scb - mrow), 0.0)
        s_sc[pl.ds(b, 1), :] = erow
        return ssum + jnp.sum(jnp.where(oh, erow, 0.0), axis=1, keepdims=True)

    ssum = lax.fori_loop(0, NR // 128, pass2, jnp.zeros((G, 1), f32))
    ssafe = jnp.where(ssum > 0.0, ssum, 1.0)

    def pass3(b, ge):
        erow = s_sc[pl.ds(b, 1), :]
        oh = oh_mask(b)
        srow = jnp.sum(jnp.where(oh, jnp.broadcast_to(ssafe, (G, 128)), 0.0),
                       axis=0, keepdims=True)
        srow = jnp.where(srow > 0.0, srow, 1.0)
        att = erow / srow
        ohw = jnp.where(oh, jnp.broadcast_to(att, (G, 128)), 0.0)
        hblk = h2_sc[pl.ds(b * 128, 128), :]
        return ge + jnp.dot(ohw, hblk, preferred_element_type=f32)

    ge = lax.fori_loop(0, NR // 128, pass3, jnp.zeros((G, D), f32))
    rtu = jnp.dot(ge, lw_ref[...], preferred_element_type=f32) + lb_ref[...]
    o_ref[...] = jax.nn.sigmoid(rtu)


_tc_readout = pl.pallas_call(
    _readout_body,
    out_shape=jax.ShapeDtypeStruct((G, 1), f32),
    scratch_shapes=[pltpu.VMEM((NR, D), f32),
                    pltpu.VMEM((NR // 128, 128), f32)],
)


def kernel(nodeTypes, edge_index, edge_attr, bs, emb, W1, root1, b1,
           W2, root2, b2, att_w, lin_w, lin_b):
    src = edge_index[0]
    dst = edge_index[1]
    nt_p = jnp.pad(nodeTypes.astype(i32), (0, NP - N))
    src_p = jnp.pad(src.astype(i32), (0, EP - E))
    dst_p = jnp.pad(dst.astype(i32), (0, EP - E), constant_values=DUMMY_DST)
    et_p = jnp.pad(edge_attr.astype(i32), (0, EP - E))
    bs2 = jnp.pad(bs.astype(i32), (0, NR - N)).reshape(NR // 128, 128)
    W1f = W1.transpose(1, 0, 2).reshape(D, R * D)
    W2f = W2.transpose(1, 0, 2).reshape(D, R * D)

    x_np, rowid, comb, cntp = _sc_prep(nt_p, emb, src_p, dst_p, et_p)
    x = x_np[:NR]
    norm = _tc_norm(cntp.reshape(2, C // 128, 128)).reshape(C)
    h1rows = _tc_h1(x, W1f).reshape(NR * R, D)
    aggp1 = _sc_layer(h1rows, rowid, comb, dst_p, norm)
    h1, h2rows = _tc_combine(x, aggp1[0], aggp1[1], root1,
                             b1.reshape(1, D), W2f)
    aggp2 = _sc_layer(h2rows.reshape(NR * R, D), rowid, comb, dst_p, norm)
    out = _tc_readout(aggp2[0], aggp2[1], h1, root2, b2.reshape(1, D),
                      att_w.reshape(D, 1).T, bs2, lin_w, lin_b.reshape(1, 1))
    return out
